# Initial kernel scaffold; baseline (speedup 1.0000x reference)
import jax
import jax.numpy as jnp
from jax.experimental import pallas as pl


def kernel(x, embedding):
    raise NotImplementedError("write your pallas kernel here")


# speedup vs baseline: 1.0606x; 1.0606x over previous
import functools
import math

import jax
import jax.numpy as jnp
from jax.experimental import pallas as pl


def _vq_kernel(x_ref, xsq_ref, esq_ref, emb_ref,
               out_ref, kl_ref, cnt_ref, perp_ref,
               *, rblks, m, inv_positions, log_m):
    n = pl.program_id(0)
    r = pl.program_id(1)

    x_blk = x_ref[0]
    xsq = xsq_ref[0]
    esq = esq_ref[0]
    emb = emb_ref[0]

    g = jax.lax.dot_general(
        x_blk, emb, (((1,), (1,)), ((), ())),
        preferred_element_type=jnp.float32)
    dist = (esq + xsq) - 2.0 * g
    logits = -dist
    mx = jnp.max(logits, axis=1, keepdims=True)
    shifted = logits - mx
    u = jnp.exp(shifted)
    s = jnp.sum(u, axis=1, keepdims=True)
    p = u / s
    log_p = shifted - jnp.log(s)

    mxp = jnp.max(p, axis=1, keepdims=True)
    iota = jax.lax.broadcasted_iota(jnp.int32, p.shape, 1)
    idx = jnp.min(jnp.where(p == mxp, iota, m), axis=1, keepdims=True)
    onehot = (iota == idx).astype(jnp.float32)

    out_ref[0] = jax.lax.dot_general(
        onehot, emb, (((1,), (0,)), ((), ())),
        preferred_element_type=jnp.float32,
        precision=jax.lax.Precision.HIGHEST)

    kl_t = p * (log_p + log_m)
    kl_t = jnp.where(p == 0.0, 0.0, kl_t)
    kl_blk = jnp.sum(kl_t, keepdims=True)

    @pl.when(jnp.logical_and(n == 0, r == 0))
    def _init_scalars():
        kl_ref[...] = jnp.zeros((1, 1), jnp.float32)
        perp_ref[...] = jnp.zeros((1, 1), jnp.float32)

    kl_ref[...] += kl_blk

    @pl.when(r == 0)
    def _init_counts():
        cnt_ref[0] = jnp.zeros_like(cnt_ref[0])

    cnt_ref[0] += jnp.sum(onehot, axis=0, keepdims=True)

    @pl.when(r == rblks - 1)
    def _perp():
        avg = cnt_ref[0] * inv_positions
        ent = jnp.sum(avg * jnp.log(avg + 1e-10), axis=1, keepdims=True)
        perp_ref[...] += jnp.exp(-ent)


def kernel(x, embedding):
    B, C, H, W = x.shape
    N, M, D = embedding.shape
    positions = B * H * W
    x_flat = x.reshape(B, N, D, H, W).transpose(1, 0, 3, 4, 2)
    x_flat = x_flat.reshape(N, positions, D)
    x_sq = jnp.sum(x_flat ** 2, axis=2, keepdims=True)
    e_sq = jnp.sum(embedding ** 2, axis=2)[:, None, :]

    rows = 128
    rblks = positions // rows
    kern = functools.partial(
        _vq_kernel, rblks=rblks, m=M,
        inv_positions=1.0 / positions, log_m=float(math.log(M)))
    out_q, kl, _cnt, perp = pl.pallas_call(
        kern,
        grid=(N, rblks),
        in_specs=[
            pl.BlockSpec((1, rows, D), lambda n, r: (n, r, 0)),
            pl.BlockSpec((1, rows, 1), lambda n, r: (n, r, 0)),
            pl.BlockSpec((1, 1, M), lambda n, r: (n, 0, 0)),
            pl.BlockSpec((1, M, D), lambda n, r: (n, 0, 0)),
        ],
        out_specs=[
            pl.BlockSpec((1, rows, D), lambda n, r: (n, r, 0)),
            pl.BlockSpec((1, 1), lambda n, r: (0, 0)),
            pl.BlockSpec((1, 1, M), lambda n, r: (n, 0, 0)),
            pl.BlockSpec((1, 1), lambda n, r: (0, 0)),
        ],
        out_shape=[
            jax.ShapeDtypeStruct((N, positions, D), jnp.float32),
            jax.ShapeDtypeStruct((1, 1), jnp.float32),
            jax.ShapeDtypeStruct((N, 1, M), jnp.float32),
            jax.ShapeDtypeStruct((1, 1), jnp.float32),
        ],
    )(x_flat, x_sq, e_sq, embedding)

    out = out_q.reshape(N, B, H, W, D).transpose(1, 0, 4, 2, 3)
    out = out.reshape(B, C, H, W)
    return out, kl[0, 0] / B, perp[0, 0]


# speedup vs baseline: 1.4947x; 1.4093x over previous
import functools
import math

import jax
import jax.numpy as jnp
from jax.experimental import pallas as pl


def _vq_kernel(x_ref, xsq_ref, esq_ref, emb_ref,
               out_ref, kl_ref, cnt_ref, perp_ref,
               *, rblks, m, inv_positions, log_m):
    n = pl.program_id(0)
    r = pl.program_id(1)

    x_blk = x_ref[0]
    xsq = xsq_ref[0]
    esq = esq_ref[0]
    emb = emb_ref[0]

    g = jax.lax.dot_general(
        x_blk, emb, (((1,), (1,)), ((), ())),
        preferred_element_type=jnp.float32)
    logits = 2.0 * g - (esq + xsq)
    mx = jnp.max(logits, axis=1, keepdims=True)
    shifted = logits - mx
    u = jnp.exp(shifted)
    s_col = u[:, 0:128]
    for i in range(1, u.shape[1] // 128):
        s_col = s_col + u[:, i * 128:(i + 1) * 128]
    b = s_col[:, 0:8]
    for k in range(1, 16):
        b = b + s_col[:, 8 * k:8 * k + 8]
    v4 = b[:, 0:4] + b[:, 4:8]
    v2 = v4[:, 0:2] + v4[:, 2:4]
    s = v2[:, 0:1] + v2[:, 1:2]
    p = u / s
    log_p = shifted - jnp.log(s)

    mxp = jnp.max(p, axis=1, keepdims=True)
    iota = jax.lax.broadcasted_iota(jnp.int32, p.shape, 1)
    idx = jnp.min(jnp.where(p == mxp, iota, m), axis=1, keepdims=True)
    onehot = (iota == idx).astype(jnp.float32)

    out_ref[0] = jax.lax.dot_general(
        onehot, emb, (((1,), (0,)), ((), ())),
        preferred_element_type=jnp.float32)

    kl_t = p * (log_p + log_m)
    kl_blk = jnp.sum(kl_t, keepdims=True)

    @pl.when(jnp.logical_and(n == 0, r == 0))
    def _init_scalars():
        kl_ref[...] = jnp.zeros((1, 1), jnp.float32)
        perp_ref[...] = jnp.zeros((1, 1), jnp.float32)

    kl_ref[...] += kl_blk

    @pl.when(r == 0)
    def _init_counts():
        cnt_ref[0] = jnp.zeros_like(cnt_ref[0])

    cnt_ref[0] += jnp.sum(onehot, axis=0, keepdims=True)

    @pl.when(r == rblks - 1)
    def _perp():
        avg = cnt_ref[0] * inv_positions
        ent = jnp.sum(avg * jnp.log(avg + 1e-10), axis=1, keepdims=True)
        perp_ref[...] += jnp.exp(-ent)


def kernel(x, embedding):
    B, C, H, W = x.shape
    N, M, D = embedding.shape
    positions = B * H * W
    x_flat = x.reshape(B, N, D, H, W).transpose(1, 0, 3, 4, 2)
    x_flat = x_flat.reshape(N, positions, D)
    x_sq = jnp.sum(x_flat ** 2, axis=2, keepdims=True)
    e_sq = jnp.sum(embedding ** 2, axis=2)[:, None, :]

    rows = 128
    rblks = positions // rows
    kern = functools.partial(
        _vq_kernel, rblks=rblks, m=M,
        inv_positions=1.0 / positions, log_m=float(math.log(M)))
    out_q, kl, _cnt, perp = pl.pallas_call(
        kern,
        grid=(N, rblks),
        in_specs=[
            pl.BlockSpec((1, rows, D), lambda n, r: (n, r, 0)),
            pl.BlockSpec((1, rows, 1), lambda n, r: (n, r, 0)),
            pl.BlockSpec((1, 1, M), lambda n, r: (n, 0, 0)),
            pl.BlockSpec((1, M, D), lambda n, r: (n, 0, 0)),
        ],
        out_specs=[
            pl.BlockSpec((1, rows, D), lambda n, r: (n, r, 0)),
            pl.BlockSpec((1, 1), lambda n, r: (0, 0)),
            pl.BlockSpec((1, 1, M), lambda n, r: (n, 0, 0)),
            pl.BlockSpec((1, 1), lambda n, r: (0, 0)),
        ],
        out_shape=[
            jax.ShapeDtypeStruct((N, positions, D), jnp.float32),
            jax.ShapeDtypeStruct((1, 1), jnp.float32),
            jax.ShapeDtypeStruct((N, 1, M), jnp.float32),
            jax.ShapeDtypeStruct((1, 1), jnp.float32),
        ],
    )(x_flat, x_sq, e_sq, embedding)

    out = out_q.reshape(N, B, H, W, D).transpose(1, 0, 4, 2, 3)
    out = out.reshape(B, C, H, W)
    return out, kl[0, 0] / B, perp[0, 0]
